# baseline (speedup 1.0000x reference)
import functools

import jax
import jax.numpy as jnp
from jax import lax
from jax.experimental import pallas as pl
from jax.experimental.pallas import tpu as pltpu
from jax.experimental.pallas import tpu_sc as plsc

L = 16


@functools.lru_cache(maxsize=None)
def _make_kernel(B, ZD, V, D, NC, NS):
    NW = NC * NS
    assert B % (8 * NW) == 0 and D % L == 0
    b_per_w = B // NW
    assert b_per_w % L == 0

    mesh = plsc.VectorSubcoreMesh(
        core_axis_name="c", subcore_axis_name="s", num_cores=NC, num_subcores=NS
    )

    @functools.partial(
        pl.kernel,
        mesh=mesh,
        out_type=jax.ShapeDtypeStruct((B, D), jnp.float32),
        scratch_types=[
            pltpu.VMEM((b_per_w,), jnp.int32),
            pltpu.VMEM((b_per_w, ZD), jnp.int32),
            pltpu.VMEM((b_per_w, D), jnp.float32),
            pltpu.SemaphoreType.DMA,
        ],
        compiler_params=pltpu.CompilerParams(needs_layout_passes=False),
    )
    def k(obs_hbm, z_hbm, table_hbm, out_hbm, obs_v, z_v, rows_v, sem):
        wid = lax.axis_index("s") * NC + lax.axis_index("c")
        base = wid * b_per_w
        pltpu.sync_copy(obs_hbm.at[pl.ds(base, b_per_w)], obs_v)
        pltpu.sync_copy(z_hbm.at[pl.ds(base, b_per_w)], z_v)
        iota = lax.iota(jnp.int32, L)
        copies = []
        for g in range(b_per_w // L):
            acc = obs_v[pl.ds(g * L, L)]
            rowi = iota + g * L
            for j in range(ZD):
                bits = plsc.load_gather(z_v, [rowi, jnp.zeros((L,), jnp.int32) + j])
                acc = acc + bits * (1 << (ZD - 1 - j))
            s = jnp.where(acc >= V, acc - V, acc)
            for k_ in range(L):
                sk = jnp.max(jnp.where(iota == k_, s, 0), axis=0)
                copies.append(
                    pltpu.make_async_copy(
                        table_hbm.at[sk], rows_v.at[g * L + k_], sem
                    )
                )
                copies[-1].start()
        for c in copies:
            c.wait()
        pltpu.sync_copy(rows_v, out_hbm.at[pl.ds(base, b_per_w)])

    return k


def kernel(obs_hash, z, seed_to_action):
    B, ZD = z.shape
    V, D = seed_to_action.shape
    info = plsc.get_sparse_core_info()
    k = _make_kernel(B, ZD, V, D, info.num_cores, info.num_subcores)
    return k(
        obs_hash.astype(jnp.int32),
        z.astype(jnp.int32),
        seed_to_action,
    )


# speedup vs baseline: 1.0830x; 1.0830x over previous
import functools

import jax
import jax.numpy as jnp
from jax import lax
from jax.experimental import pallas as pl
from jax.experimental.pallas import tpu as pltpu
from jax.experimental.pallas import tpu_sc as plsc

L = 16


@functools.lru_cache(maxsize=None)
def _make_kernel(B, ZD, V, D, NC, NS):
    NW = NC * NS
    assert B % (8 * NW) == 0 and D % L == 0
    b_per_w = B // NW
    assert b_per_w % L == 0

    mesh = plsc.VectorSubcoreMesh(
        core_axis_name="c", subcore_axis_name="s", num_cores=NC, num_subcores=NS
    )

    @functools.partial(
        pl.kernel,
        mesh=mesh,
        out_type=jax.ShapeDtypeStruct((B, D), jnp.float32),
        scratch_types=[
            pltpu.VMEM((b_per_w,), jnp.int32),
            pltpu.VMEM((ZD, b_per_w), jnp.int32),
            pltpu.VMEM((b_per_w, D), jnp.float32),
            pltpu.SemaphoreType.DMA,
        ],
        compiler_params=pltpu.CompilerParams(needs_layout_passes=False),
    )
    def k(obs_hbm, zt_hbm, table_hbm, out_hbm, obs_v, z_v, rows_v, sem):
        wid = lax.axis_index("s") * NC + lax.axis_index("c")
        base = wid * b_per_w
        pltpu.sync_copy(obs_hbm.at[pl.ds(base, b_per_w)], obs_v)
        pltpu.sync_copy(zt_hbm.at[:, pl.ds(base, b_per_w)], z_v)
        iota = lax.iota(jnp.int32, L)
        for g in range(b_per_w // L):
            acc = obs_v[pl.ds(g * L, L)]
            for j in range(ZD):
                acc = acc + z_v[j, pl.ds(g * L, L)] * (1 << (ZD - 1 - j))
            s = jnp.where(acc >= V, acc - V, acc)

            def fire(k_, carry, s=s, g=g):
                sk = jnp.max(jnp.where(iota == k_, s, 0), axis=0)
                pltpu.make_async_copy(
                    table_hbm.at[sk], rows_v.at[g * L + k_], sem
                ).start()
                return carry

            lax.fori_loop(0, L, fire, 0)
        pltpu.make_async_copy(
            table_hbm.at[pl.ds(0, b_per_w)], rows_v, sem
        ).wait()
        pltpu.sync_copy(rows_v, out_hbm.at[pl.ds(base, b_per_w)])

    return k


def kernel(obs_hash, z, seed_to_action):
    B, ZD = z.shape
    V, D = seed_to_action.shape
    info = plsc.get_sparse_core_info()
    k = _make_kernel(B, ZD, V, D, info.num_cores, info.num_subcores)
    return k(
        obs_hash.astype(jnp.int32),
        z.astype(jnp.int32).T,
        seed_to_action,
    )
